# baseline (speedup 1.0000x reference)
import jax
import jax.numpy as jnp
from jax import lax
from jax.experimental import pallas as pl
from jax.experimental.pallas import tpu as pltpu
from jax.experimental.pallas import tpu_sc as plsc

N = 10000
E = 160000
H = 8
DK = 32
ROW = H * DK

NC = 2
NS = 16

HALF = N // NC
ZBASE = 10048
ACC_ROWS = 10368
ZERO_CHUNK = ACC_ROWS // NS
C = 32
NIDX = (C + 15) // 16
INV_SQRT_DK = 1.0 / float(DK) ** 0.5


def _body(qf, kf, vf, src, dst, seg, zsrc, out,
          srcb, dstb, dstb48, idxb, zidxb, zbaseb, segb,
          kb, qb, vb, vs, acc, sem):
  c = lax.axis_index("c")
  s = lax.axis_index("s")

  pltpu.sync_copy(zsrc.at[pl.ds(s * ZERO_CHUNK, ZERO_CHUNK)],
                  acc.at[pl.ds(s * ZERO_CHUNK, ZERO_CHUNK)])
  pltpu.sync_copy(seg, segb)
  plsc.subcore_barrier()

  dst_base = c * HALF
  lanes = lax.iota(jnp.int32, 16)
  zero16 = jnp.zeros((16,), jnp.float32)

  bstart = segb[pl.ds(2 * c, 16)][0]
  bend = segb[pl.ds(2 * c + 1, 16)][0]
  nblk = jnp.maximum(0, bend - bstart - s + NS - 1) // NS

  def edge_fn_p(e):
    zv_lo = zero16
    zv_hi = zero16
    for h in range(H):
      a0 = kb[e, pl.ds(32 * h, 16)] * qb[e, pl.ds(32 * h, 16)]
      a1 = kb[e, pl.ds(32 * h + 16, 16)] * qb[e, pl.ds(32 * h + 16, 16)]
      tot = jnp.sum(a0 + a1)
      sc = jnp.minimum(jnp.maximum(tot * INV_SQRT_DK, -5.0), 5.0)
      ev = jnp.exp(jnp.broadcast_to(sc, (16,)))
      zv_lo = jnp.where(lanes == h, ev, zv_lo)
      zv_hi = jnp.where(lanes == h + 8, ev, zv_hi)
      for t in range(2):
        col = 32 * h + 16 * t
        vs[col // 128, e, pl.ds(col % 128, 16)] = vb[e, pl.ds(col, 16)] * ev
    zpk = zbaseb[pl.ds(e, 16)][0]
    zb = zpk & ~15
    hi = zpk & 8
    vs[2, e, pl.ds(zb, 16)] = jnp.where(hi > 0, zv_hi, zv_lo)

  def blk_fn(b, _):
    e0 = (bstart + s + b * NS) * C
    pltpu.sync_copy(src.at[pl.ds(e0, C)], srcb)
    pltpu.sync_copy(dst.at[pl.ds(e0, C)], dstb)
    pltpu.sync_copy(dst.at[pl.ds(e0, 16 * NIDX)], dstb48)
    d1 = pltpu.async_copy(kf.at[srcb], kb, sem)
    d2 = pltpu.async_copy(qf.at[dstb], qb, sem)
    d3 = pltpu.async_copy(vf.at[srcb], vb, sem)
    d4 = pltpu.async_copy(zsrc.at[pl.ds(0, C)], vs.at[2], sem)

    for i in range(NIDX):
      lanes_g = 16 * i + lanes
      valid = lanes_g < C
      dv = dstb48[pl.ds(16 * i, 16)]
      loc = dv - dst_base
      ok = (loc >= 0) & (loc < HALF)
      locs = jnp.where(ok & valid, loc, HALF)
      for j in range(2):
        plsc.store_scatter(idxb, [jnp.full((16,), j, jnp.int32), lanes_g],
                           2 * locs + j, mask=valid)
      plsc.store_scatter(zidxb, [lanes_g], ZBASE + (locs >> 4), mask=valid)
      plsc.store_scatter(zbaseb, [lanes_g], (locs & 15) << 3, mask=valid)

    d1.wait()
    d2.wait()
    d3.wait()
    d4.wait()

    plsc.parallel_loop(0, C, unroll=2)(edge_fn_p)

    for j in range(2):
      pltpu.sync_copy(vs.at[j], acc.at[idxb.at[j]], add=True)
    pltpu.sync_copy(vs.at[2], acc.at[zidxb], add=True)
    return 0

  lax.fori_loop(0, nblk, blk_fn, 0)
  plsc.subcore_barrier()

  r0 = s * ZERO_CHUNK
  pltpu.sync_copy(acc.at[pl.ds(r0, ZERO_CHUNK)],
                  out.at[pl.ds(c * ACC_ROWS + r0, ZERO_CHUNK)])


@jax.jit
def _run(qf, kf, vf, src, dst, seg, zsrc):
  mesh = plsc.VectorSubcoreMesh(core_axis_name="c", subcore_axis_name="s",
                                num_cores=NC, num_subcores=NS)
  return pl.kernel(
      _body,
      out_type=jax.ShapeDtypeStruct((NC * ACC_ROWS, 128), jnp.float32),
      mesh=mesh,
      compiler_params=pltpu.CompilerParams(needs_layout_passes=False),
      scratch_types=[
          pltpu.VMEM((C,), jnp.int32),
          pltpu.VMEM((C,), jnp.int32),
          pltpu.VMEM((16 * NIDX,), jnp.int32),
          pltpu.VMEM((2, C), jnp.int32),
          pltpu.VMEM((C,), jnp.int32),
          pltpu.VMEM((C + 16,), jnp.int32),
          pltpu.VMEM((32,), jnp.int32),
          pltpu.VMEM((C, ROW), jnp.float32),
          pltpu.VMEM((C, ROW), jnp.float32),
          pltpu.VMEM((C, ROW), jnp.float32),
          pltpu.VMEM((3, C, 128), jnp.float32),
          pltpu.VMEM_SHARED((ACC_ROWS, 128), jnp.float32),
          pltpu.SemaphoreType.DMA,
      ],
  )(qf, kf, vf, src, dst, seg, zsrc)


def kernel(q, k, v, edge_index):
  qf = q.reshape(N, ROW)
  kf = k.reshape(N, ROW)
  vf = v.reshape(N, ROW)
  src0 = edge_index[0]
  dst0 = edge_index[1]
  in0 = (dst0 < HALF).astype(jnp.int32)
  csum = jnp.cumsum(in0)
  count0 = csum[E - 1]
  above = count0 + (jnp.arange(E, dtype=jnp.int32) + 1 - csum) - 1
  pos = jnp.where(in0 == 1, csum - 1, above)
  sd = jnp.zeros((E + 48, 2), jnp.int32).at[pos].set(
      jnp.stack([src0, dst0], axis=1), unique_indices=True, mode="drop")
  src = sd[:, 0]
  dst = sd[:, 1]
  seg = (jnp.zeros((32,), jnp.int32)
         .at[1].set((count0 + C - 1) // C)
         .at[2].set(count0 // C)
         .at[3].set(E // C))
  zsrc = jnp.zeros((ACC_ROWS, 128), jnp.float32)
  out = _run(qf, kf, vf, src, dst, seg, zsrc)
  wv_halves = []
  z_halves = []
  for c in range(NC):
    oc = out[c * ACC_ROWS:(c + 1) * ACC_ROWS]
    wv_halves.append(oc[:2 * HALF].reshape(HALF, ROW))
    z_halves.append(oc[ZBASE:ZBASE + 313].reshape(313 * 16, 8)[:HALF])
  wv = jnp.concatenate(wv_halves, axis=0).reshape(N, H, DK)
  z = jnp.concatenate(z_halves, axis=0).reshape(N, H, 1)
  return wv, z


# speedup vs baseline: 1.3009x; 1.3009x over previous
import jax
import jax.numpy as jnp
from jax import lax
from jax.experimental import pallas as pl
from jax.experimental.pallas import tpu as pltpu
from jax.experimental.pallas import tpu_sc as plsc

N = 10000
E = 160000
H = 8
DK = 32
ROW = H * DK

NC = 2
NS = 16

HALF = N // NC
ZBASE = 10048
ACC_ROWS = 10368
ZERO_CHUNK = ACC_ROWS // NS
C = 32
NIDX = (C + 15) // 16
INV_SQRT_DK = 1.0 / float(DK) ** 0.5


def _body(qf, kf, vf, src, dst, seg, zsrc, out,
          srcb, dstb, dstb48, idxb, zidxb, zbaseb, segb,
          kb, qb, vb, vs, acc, sem):
  c = lax.axis_index("c")
  s = lax.axis_index("s")

  pltpu.sync_copy(zsrc.at[pl.ds(s * ZERO_CHUNK, ZERO_CHUNK)],
                  acc.at[pl.ds(s * ZERO_CHUNK, ZERO_CHUNK)])
  pltpu.sync_copy(seg, segb)
  plsc.subcore_barrier()

  dst_base = c * HALF
  lanes = lax.iota(jnp.int32, 16)
  zero16 = jnp.zeros((16,), jnp.float32)

  bstart = segb[pl.ds(2 * c, 16)][0]
  bend = segb[pl.ds(2 * c + 1, 16)][0]
  nblk = jnp.maximum(0, bend - bstart - s + NS - 1) // NS

  def edge_fn_p(e):
    zv_lo = zero16
    zv_hi = zero16
    for h in range(H):
      a0 = kb[e, pl.ds(32 * h, 16)] * qb[e, pl.ds(32 * h, 16)]
      a1 = kb[e, pl.ds(32 * h + 16, 16)] * qb[e, pl.ds(32 * h + 16, 16)]
      tot = jnp.sum(a0 + a1)
      sc = jnp.minimum(jnp.maximum(tot * INV_SQRT_DK, -5.0), 5.0)
      ev = jnp.exp(jnp.broadcast_to(sc, (16,)))
      zv_lo = jnp.where(lanes == h, ev, zv_lo)
      zv_hi = jnp.where(lanes == h + 8, ev, zv_hi)
      for t in range(2):
        col = 32 * h + 16 * t
        vs[col // 128, e, pl.ds(col % 128, 16)] = vb[e, pl.ds(col, 16)] * ev
    zpk = zbaseb[pl.ds(e, 16)][0]
    zb = zpk & ~15
    hi = zpk & 8
    vs[2, e, pl.ds(zb, 16)] = jnp.where(hi > 0, zv_hi, zv_lo)

  def blk_fn(b, _):
    e0 = (bstart + s + b * NS) * C
    pltpu.sync_copy(src.at[pl.ds(e0, C)], srcb)
    pltpu.sync_copy(dst.at[pl.ds(e0, C)], dstb)
    pltpu.sync_copy(dst.at[pl.ds(e0, 16 * NIDX)], dstb48)
    d1 = pltpu.async_copy(kf.at[srcb], kb, sem)
    d2 = pltpu.async_copy(qf.at[dstb], qb, sem)
    d3 = pltpu.async_copy(vf.at[srcb], vb, sem)
    d4 = pltpu.async_copy(zsrc.at[pl.ds(0, C)], vs.at[2], sem)

    for i in range(NIDX):
      lanes_g = 16 * i + lanes
      valid = lanes_g < C
      dv = dstb48[pl.ds(16 * i, 16)]
      loc = dv - dst_base
      ok = (loc >= 0) & (loc < HALF)
      locs = jnp.where(ok & valid, loc, HALF)
      for j in range(2):
        plsc.store_scatter(idxb, [jnp.full((16,), j, jnp.int32), lanes_g],
                           2 * locs + j, mask=valid)
      plsc.store_scatter(zidxb, [lanes_g], ZBASE + (locs >> 4), mask=valid)
      plsc.store_scatter(zbaseb, [lanes_g], (locs & 15) << 3, mask=valid)

    d1.wait()
    d2.wait()
    d3.wait()
    d4.wait()

    plsc.parallel_loop(0, C, unroll=2)(edge_fn_p)

    for j in range(2):
      pltpu.sync_copy(vs.at[j], acc.at[idxb.at[j]], add=True)
    pltpu.sync_copy(vs.at[2], acc.at[zidxb], add=True)
    return 0

  lax.fori_loop(0, nblk, blk_fn, 0)
  plsc.subcore_barrier()

  r0 = s * ZERO_CHUNK
  pltpu.sync_copy(acc.at[pl.ds(r0, ZERO_CHUNK)],
                  out.at[pl.ds(c * ACC_ROWS + r0, ZERO_CHUNK)])


@jax.jit
def _run(qf, kf, vf, src, dst, seg, zsrc):
  mesh = plsc.VectorSubcoreMesh(core_axis_name="c", subcore_axis_name="s",
                                num_cores=NC, num_subcores=NS)
  return pl.kernel(
      _body,
      out_type=jax.ShapeDtypeStruct((NC * ACC_ROWS, 128), jnp.float32),
      mesh=mesh,
      compiler_params=pltpu.CompilerParams(needs_layout_passes=False),
      scratch_types=[
          pltpu.VMEM((C,), jnp.int32),
          pltpu.VMEM((C,), jnp.int32),
          pltpu.VMEM((16 * NIDX,), jnp.int32),
          pltpu.VMEM((2, C), jnp.int32),
          pltpu.VMEM((C,), jnp.int32),
          pltpu.VMEM((C + 16,), jnp.int32),
          pltpu.VMEM((32,), jnp.int32),
          pltpu.VMEM((C, ROW), jnp.float32),
          pltpu.VMEM((C, ROW), jnp.float32),
          pltpu.VMEM((C, ROW), jnp.float32),
          pltpu.VMEM((3, C, 128), jnp.float32),
          pltpu.VMEM_SHARED((ACC_ROWS, 128), jnp.float32),
          pltpu.SemaphoreType.DMA,
      ],
  )(qf, kf, vf, src, dst, seg, zsrc)


def kernel(q, k, v, edge_index):
  qf = q.reshape(N, ROW)
  kf = k.reshape(N, ROW)
  vf = v.reshape(N, ROW)
  src0 = edge_index[0]
  dst0 = edge_index[1]
  in0 = (dst0 < HALF).astype(jnp.int32)
  csum = jnp.cumsum(in0)
  count0 = csum[E - 1]
  above = count0 + (jnp.arange(E, dtype=jnp.int32) + 1 - csum) - 1
  pos = jnp.where(in0 == 1, csum - 1, above)
  src = jnp.zeros((E + 48,), jnp.int32).at[pos].set(
      src0, unique_indices=True, mode="drop")
  dst = jnp.zeros((E + 48,), jnp.int32).at[pos].set(
      dst0, unique_indices=True, mode="drop")
  seg = (jnp.zeros((32,), jnp.int32)
         .at[1].set((count0 + C - 1) // C)
         .at[2].set(count0 // C)
         .at[3].set(E // C))
  zsrc = jnp.zeros((ACC_ROWS, 128), jnp.float32)
  out = _run(qf, kf, vf, src, dst, seg, zsrc)
  wv_halves = []
  z_halves = []
  for c in range(NC):
    oc = out[c * ACC_ROWS:(c + 1) * ACC_ROWS]
    wv_halves.append(oc[:2 * HALF].reshape(HALF, ROW))
    z_halves.append(oc[ZBASE:ZBASE + 313].reshape(313 * 16, 8)[:HALF])
  wv = jnp.concatenate(wv_halves, axis=0).reshape(N, H, DK)
  z = jnp.concatenate(z_halves, axis=0).reshape(N, H, 1)
  return wv, z


# speedup vs baseline: 2.2547x; 1.7332x over previous
import jax
import jax.numpy as jnp
from jax import lax
from jax.experimental import pallas as pl
from jax.experimental.pallas import tpu as pltpu
from jax.experimental.pallas import tpu_sc as plsc

N = 10000
E = 160000
H = 8
DK = 32
ROW = H * DK

NC = 2
NS = 16

HALF = N // NC
ZBASE = 10048
ACC_ROWS = 10368
ZERO_CHUNK = ACC_ROWS // NS
C = 32
NIDX = (C + 15) // 16
INV_SQRT_DK = 1.0 / float(DK) ** 0.5


def _body(qf, kf, vf, src, dst, seg, zsrc, out,
          srcb, dstb, dstb48, idxb, zidxb, zbaseb, segb,
          kb, qb, vb, vs, acc, sem):
  c = lax.axis_index("c")
  s = lax.axis_index("s")

  pltpu.sync_copy(zsrc.at[pl.ds(s * ZERO_CHUNK, ZERO_CHUNK)],
                  acc.at[pl.ds(s * ZERO_CHUNK, ZERO_CHUNK)])
  pltpu.sync_copy(seg, segb)
  plsc.subcore_barrier()

  dst_base = c * HALF
  lanes = lax.iota(jnp.int32, 16)
  zero16 = jnp.zeros((16,), jnp.float32)

  bstart = segb[pl.ds(2 * c, 16)][0]
  bend = segb[pl.ds(2 * c + 1, 16)][0]
  nblk = jnp.maximum(0, bend - bstart - s + NS - 1) // NS

  def edge_fn_p(e):
    zv_lo = zero16
    zv_hi = zero16
    for h in range(H):
      a0 = kb[e, pl.ds(32 * h, 16)] * qb[e, pl.ds(32 * h, 16)]
      a1 = kb[e, pl.ds(32 * h + 16, 16)] * qb[e, pl.ds(32 * h + 16, 16)]
      tot = jnp.sum(a0 + a1)
      sc = jnp.minimum(jnp.maximum(tot * INV_SQRT_DK, -5.0), 5.0)
      ev = jnp.exp(jnp.broadcast_to(sc, (16,)))
      zv_lo = jnp.where(lanes == h, ev, zv_lo)
      zv_hi = jnp.where(lanes == h + 8, ev, zv_hi)
      for t in range(2):
        col = 32 * h + 16 * t
        vs[col // 128, e, pl.ds(col % 128, 16)] = vb[e, pl.ds(col, 16)] * ev
    zpk = zbaseb[pl.ds(e, 16)][0]
    zb = zpk & ~15
    hi = zpk & 8
    vs[2, e, pl.ds(zb, 16)] = jnp.where(hi > 0, zv_hi, zv_lo)

  def blk_fn(b, _):
    e0 = (bstart + s + b * NS) * C
    pltpu.sync_copy(src.at[pl.ds(e0, C)], srcb)
    pltpu.sync_copy(dst.at[pl.ds(e0, C)], dstb)
    pltpu.sync_copy(dst.at[pl.ds(e0, 16 * NIDX)], dstb48)
    d1 = pltpu.async_copy(kf.at[srcb], kb, sem)
    d2 = pltpu.async_copy(qf.at[dstb], qb, sem)
    d3 = pltpu.async_copy(vf.at[srcb], vb, sem)
    d4 = pltpu.async_copy(zsrc.at[pl.ds(0, C)], vs.at[2], sem)

    for i in range(NIDX):
      lanes_g = 16 * i + lanes
      valid = lanes_g < C
      dv = dstb48[pl.ds(16 * i, 16)]
      loc = dv - dst_base
      ok = (loc >= 0) & (loc < HALF)
      locs = jnp.where(ok & valid, loc, HALF)
      for j in range(2):
        plsc.store_scatter(idxb, [jnp.full((16,), j, jnp.int32), lanes_g],
                           2 * locs + j, mask=valid)
      plsc.store_scatter(zidxb, [lanes_g], ZBASE + (locs >> 4), mask=valid)
      plsc.store_scatter(zbaseb, [lanes_g], (locs & 15) << 3, mask=valid)

    d1.wait()
    d2.wait()
    d3.wait()
    d4.wait()

    plsc.parallel_loop(0, C, unroll=2)(edge_fn_p)

    for j in range(2):
      pltpu.sync_copy(vs.at[j], acc.at[idxb.at[j]], add=True)
    pltpu.sync_copy(vs.at[2], acc.at[zidxb], add=True)
    return 0

  lax.fori_loop(0, nblk, blk_fn, 0)
  plsc.subcore_barrier()

  r0 = s * ZERO_CHUNK
  pltpu.sync_copy(acc.at[pl.ds(r0, ZERO_CHUNK)],
                  out.at[pl.ds(c * ACC_ROWS + r0, ZERO_CHUNK)])


@jax.jit
def _run(qf, kf, vf, src, dst, seg, zsrc):
  mesh = plsc.VectorSubcoreMesh(core_axis_name="c", subcore_axis_name="s",
                                num_cores=NC, num_subcores=NS)
  return pl.kernel(
      _body,
      out_type=jax.ShapeDtypeStruct((NC * ACC_ROWS, 128), jnp.float32),
      mesh=mesh,
      compiler_params=pltpu.CompilerParams(needs_layout_passes=False),
      scratch_types=[
          pltpu.VMEM((C,), jnp.int32),
          pltpu.VMEM((C,), jnp.int32),
          pltpu.VMEM((16 * NIDX,), jnp.int32),
          pltpu.VMEM((2, C), jnp.int32),
          pltpu.VMEM((C,), jnp.int32),
          pltpu.VMEM((C + 16,), jnp.int32),
          pltpu.VMEM((32,), jnp.int32),
          pltpu.VMEM((C, ROW), jnp.float32),
          pltpu.VMEM((C, ROW), jnp.float32),
          pltpu.VMEM((C, ROW), jnp.float32),
          pltpu.VMEM((3, C, 128), jnp.float32),
          pltpu.VMEM_SHARED((ACC_ROWS, 128), jnp.float32),
          pltpu.SemaphoreType.DMA,
      ],
  )(qf, kf, vf, src, dst, seg, zsrc)


def kernel(q, k, v, edge_index):
  qf = q.reshape(N, ROW)
  kf = k.reshape(N, ROW)
  vf = v.reshape(N, ROW)
  src0 = edge_index[0]
  dst0 = edge_index[1]
  key = (dst0 >= HALF).astype(jnp.int32)
  count0 = E - jnp.sum(key)
  perm = jnp.argsort(key, stable=True)
  pad = jnp.zeros((48,), jnp.int32)
  src = jnp.concatenate([src0[perm], pad])
  dst = jnp.concatenate([dst0[perm], pad])
  seg = (jnp.zeros((32,), jnp.int32)
         .at[1].set((count0 + C - 1) // C)
         .at[2].set(count0 // C)
         .at[3].set(E // C))
  zsrc = jnp.zeros((ACC_ROWS, 128), jnp.float32)
  out = _run(qf, kf, vf, src, dst, seg, zsrc)
  wv_halves = []
  z_halves = []
  for c in range(NC):
    oc = out[c * ACC_ROWS:(c + 1) * ACC_ROWS]
    wv_halves.append(oc[:2 * HALF].reshape(HALF, ROW))
    z_halves.append(oc[ZBASE:ZBASE + 313].reshape(313 * 16, 8)[:HALF])
  wv = jnp.concatenate(wv_halves, axis=0).reshape(N, H, DK)
  z = jnp.concatenate(z_halves, axis=0).reshape(N, H, 1)
  return wv, z
